# Initial kernel scaffold; baseline (speedup 1.0000x reference)
import jax
import jax.numpy as jnp
from jax.experimental import pallas as pl


def kernel(sparse_indices, dense_feature, tables):
    raise NotImplementedError("write your pallas kernel here")


# speedup vs baseline: 4.0009x; 4.0009x over previous
import functools

import jax
import jax.numpy as jnp
from jax import lax
from jax.experimental import pallas as pl
from jax.experimental.pallas import tpu as pltpu
from jax.experimental.pallas import tpu_sc as plsc

F, B, L, V, D = 26, 4096, 20, 100000, 16
NC, NS = 2, 16
NW = NC * NS
BPW = B // NW
IPW = BPW * L
GCH = 128
NG = IPW // GCH


def _sc_body(tables_hbm, idx_hbm, out_hbm, idx_v, rows_v, acc_v, sem):
    wid = lax.axis_index("s") * NC + lax.axis_index("c")
    base = wid * BPW

    def feat_body(f, carry):
        pltpu.sync_copy(idx_hbm.at[f, pl.ds(base * L, IPW)], idx_v)

        def fire(g, c):
            pltpu.async_copy(
                tables_hbm.at[idx_v.at[pl.ds(g * GCH, GCH)]],
                rows_v.at[pl.ds(g * GCH, GCH)], sem)
            return c

        lax.fori_loop(0, NG, fire, 0)

        def drain(g, c):
            pltpu.make_async_copy(
                tables_hbm.at[idx_v.at[pl.ds(g * GCH, GCH)]],
                rows_v.at[pl.ds(g * GCH, GCH)], sem).wait()
            return c

        lax.fori_loop(0, NG, drain, 0)

        col = f * D

        def bag_body(b, c):
            r = b * L
            acc = rows_v[r, :]
            for l in range(1, L):
                acc = acc + rows_v[r + l, :]
            acc_v[b, pl.ds(col, D)] = acc
            return c

        lax.fori_loop(0, BPW, bag_body, 0)
        return carry

    lax.fori_loop(0, F, feat_body, 0)
    pltpu.sync_copy(acc_v, out_hbm.at[pl.ds(base, BPW), :])


_sc_call = functools.partial(
    pl.kernel,
    out_type=jax.ShapeDtypeStruct((B, F * D), jnp.float32),
    mesh=plsc.VectorSubcoreMesh(core_axis_name="c", subcore_axis_name="s"),
    scratch_types=[
        pltpu.VMEM((IPW,), jnp.int32),
        pltpu.VMEM((IPW, D), jnp.float32),
        pltpu.VMEM((BPW, F * D), jnp.float32),
        pltpu.SemaphoreType.DMA,
    ],
    compiler_params=pltpu.CompilerParams(use_tc_tiling_on_sc=False),
)(_sc_body)


def kernel(sparse_indices, dense_feature, tables):
    offs = (jnp.arange(F, dtype=jnp.int32) * V)[:, None]
    idx_flat = sparse_indices.reshape(F, B * L) + offs
    tables_flat = tables.reshape(F * V, D)
    sparse_out = _sc_call(tables_flat, idx_flat)
    return jnp.concatenate([sparse_out, dense_feature], axis=1)


# speedup vs baseline: 4.0132x; 1.0031x over previous
import functools

import jax
import jax.numpy as jnp
from jax import lax
from jax.experimental import pallas as pl
from jax.experimental.pallas import tpu as pltpu
from jax.experimental.pallas import tpu_sc as plsc

F, B, L, V, D = 26, 4096, 20, 100000, 16
NC, NS = 2, 16
NW = NC * NS
BPW = B // NW
IPW = BPW * L
GCH = 128
NG = IPW // GCH


def _sc_body(tables_hbm, idx_hbm, out_hbm, idx_v, rows_v, acc_v, sem):
    wid = lax.axis_index("s") * NC + lax.axis_index("c")
    base = wid * BPW

    def feat_body(f, carry):
        pltpu.sync_copy(idx_hbm.at[f, pl.ds(base * L, IPW)], idx_v)

        def fire(g, c):
            pltpu.async_copy(
                tables_hbm.at[f].at[idx_v.at[pl.ds(g * GCH, GCH)]],
                rows_v.at[pl.ds(g * GCH, GCH)], sem)
            return c

        lax.fori_loop(0, NG, fire, 0)

        def drain(g, c):
            pltpu.make_async_copy(
                tables_hbm.at[f].at[idx_v.at[pl.ds(g * GCH, GCH)]],
                rows_v.at[pl.ds(g * GCH, GCH)], sem).wait()
            return c

        lax.fori_loop(0, NG, drain, 0)

        col = f * D

        def bag_body(b, c):
            r = b * L
            acc = rows_v[r, :]
            for l in range(1, L):
                acc = acc + rows_v[r + l, :]
            acc_v[b, pl.ds(col, D)] = acc
            return c

        lax.fori_loop(0, BPW, bag_body, 0)
        return carry

    lax.fori_loop(0, F, feat_body, 0)
    pltpu.sync_copy(acc_v, out_hbm.at[pl.ds(base, BPW), :])


_sc_call = functools.partial(
    pl.kernel,
    out_type=jax.ShapeDtypeStruct((B, F * D), jnp.float32),
    mesh=plsc.VectorSubcoreMesh(core_axis_name="c", subcore_axis_name="s"),
    scratch_types=[
        pltpu.VMEM((IPW,), jnp.int32),
        pltpu.VMEM((IPW, D), jnp.float32),
        pltpu.VMEM((BPW, F * D), jnp.float32),
        pltpu.SemaphoreType.DMA,
    ],
    compiler_params=pltpu.CompilerParams(use_tc_tiling_on_sc=False),
)(_sc_body)


def kernel(sparse_indices, dense_feature, tables):
    idx_flat = sparse_indices.reshape(F, B * L)
    sparse_out = _sc_call(tables, idx_flat)
    return jnp.concatenate([sparse_out, dense_feature], axis=1)


# speedup vs baseline: 9.1149x; 2.2712x over previous
import functools

import jax
import jax.numpy as jnp
from jax import lax
from jax.experimental import pallas as pl
from jax.experimental.pallas import tpu as pltpu
from jax.experimental.pallas import tpu_sc as plsc

F, B, L, V, D = 26, 4096, 20, 100000, 16
NC, NS, NL = 2, 16, 16
NW = NC * NS
NP = F * D
PPW = NP // NW
BCH = 512
NBC = B // BCH
NBG = BCH // NL


def _sc_body(tab_hbm, idx_hbm, out_hbm, tile_v, idxc_v, out_v):
    wid = lax.axis_index("s") * NC + lax.axis_index("c")

    def pair_body(k, carry):
        p = wid * PPW + k
        f = p // D
        pltpu.sync_copy(tab_hbm.at[p], tile_v)

        def chunk_body(c, carry2):
            pltpu.sync_copy(
                idx_hbm.at[pl.ds(f * L, L), pl.ds(c * BCH, BCH)], idxc_v)

            def group_body(g, carry3):
                acc = jnp.zeros((NL,), jnp.float32)
                for l in range(L):
                    idx16 = idxc_v[l, pl.ds(g * NL, NL)]
                    acc = acc + plsc.load_gather(tile_v, [idx16])
                out_v[pl.ds(c * BCH + g * NL, NL)] = acc
                return carry3

            lax.fori_loop(0, NBG, group_body, 0)
            return carry2

        lax.fori_loop(0, NBC, chunk_body, 0)
        pltpu.sync_copy(out_v, out_hbm.at[p])
        return carry

    lax.fori_loop(0, PPW, pair_body, 0)


_sc_call = functools.partial(
    pl.kernel,
    out_type=jax.ShapeDtypeStruct((NP, B), jnp.float32),
    mesh=plsc.VectorSubcoreMesh(core_axis_name="c", subcore_axis_name="s"),
    scratch_types=[
        pltpu.VMEM((V,), jnp.float32),
        pltpu.VMEM((L, BCH), jnp.int32),
        pltpu.VMEM((B,), jnp.float32),
    ],
    compiler_params=pltpu.CompilerParams(
        use_tc_tiling_on_sc=False, needs_layout_passes=False),
)(_sc_body)


def kernel(sparse_indices, dense_feature, tables):
    tab_t = tables.transpose(0, 2, 1).reshape(F * D, V)
    idx_t = sparse_indices.transpose(0, 2, 1).reshape(F * L, B)
    pooled_t = _sc_call(tab_t, idx_t)
    sparse_out = pooled_t.T
    return jnp.concatenate([sparse_out, dense_feature], axis=1)


# speedup vs baseline: 9.6491x; 1.0586x over previous
import functools

import jax
import jax.numpy as jnp
from jax import lax
from jax.experimental import pallas as pl
from jax.experimental.pallas import tpu as pltpu
from jax.experimental.pallas import tpu_sc as plsc

F, B, L, V, D = 26, 4096, 20, 100000, 16
NC, NS, NL = 2, 16, 16
NW = NC * NS
NP = F * D
PPW = NP // NW
BCH = 512
NBC = B // BCH
NBG = BCH // NL


def _sc_body(tab_hbm, idx_hbm, out_hbm, tile_v, idxc_v, out_v):
    wid = lax.axis_index("s") * NC + lax.axis_index("c")

    def pair_body(k, carry):
        p = wid * PPW + k
        f = p // D
        pltpu.sync_copy(tab_hbm.at[p], tile_v)

        def chunk_body(c, carry2):
            pltpu.sync_copy(
                idx_hbm.at[pl.ds(f * L, L), pl.ds(c * BCH, BCH)], idxc_v)

            @plsc.parallel_loop(0, NBG, 1, unroll=2)
            def group_body(g):
                acc0 = jnp.zeros((NL,), jnp.float32)
                acc1 = jnp.zeros((NL,), jnp.float32)
                for l in range(0, L, 2):
                    i0 = idxc_v[l, pl.ds(g * NL, NL)]
                    i1 = idxc_v[l + 1, pl.ds(g * NL, NL)]
                    acc0 = acc0 + plsc.load_gather(tile_v, [i0])
                    acc1 = acc1 + plsc.load_gather(tile_v, [i1])
                out_v[pl.ds(c * BCH + g * NL, NL)] = acc0 + acc1

            return carry2

        lax.fori_loop(0, NBC, chunk_body, 0)
        pltpu.sync_copy(out_v, out_hbm.at[p])
        return carry

    lax.fori_loop(0, PPW, pair_body, 0)


_sc_call = functools.partial(
    pl.kernel,
    out_type=jax.ShapeDtypeStruct((NP, B), jnp.float32),
    mesh=plsc.VectorSubcoreMesh(core_axis_name="c", subcore_axis_name="s"),
    scratch_types=[
        pltpu.VMEM((V,), jnp.float32),
        pltpu.VMEM((L, BCH), jnp.int32),
        pltpu.VMEM((B,), jnp.float32),
    ],
    compiler_params=pltpu.CompilerParams(
        use_tc_tiling_on_sc=False, needs_layout_passes=False),
)(_sc_body)


def kernel(sparse_indices, dense_feature, tables):
    tab_t = tables.transpose(0, 2, 1).reshape(F * D, V)
    idx_t = sparse_indices.transpose(0, 2, 1).reshape(F * L, B)
    pooled_t = _sc_call(tab_t, idx_t)
    sparse_out = pooled_t.T
    return jnp.concatenate([sparse_out, dense_feature], axis=1)


# speedup vs baseline: 10.5988x; 1.0984x over previous
import functools

import jax
import jax.numpy as jnp
from jax import lax
from jax.experimental import pallas as pl
from jax.experimental.pallas import tpu as pltpu
from jax.experimental.pallas import tpu_sc as plsc

F, B, L, V, D = 26, 4096, 20, 100000, 16
NC, NS, NL = 2, 16, 16
NW = NC * NS
NP = F * D
BCH = 512
NBC = B // BCH
NBG = BCH // NL
SPLITS = (96, 96, 96, 128)


def _make_call(start, npairs):
    ppw = npairs // NW

    def body(tab_hbm, idx_hbm, out_hbm, tile_v, idxc_v, out_v):
        wid = lax.axis_index("s") * NC + lax.axis_index("c")

        def pair_body(k, carry):
            p = wid * ppw + k
            f = (start + p) // D
            pltpu.sync_copy(tab_hbm.at[p], tile_v)

            def chunk_body(c, carry2):
                pltpu.sync_copy(
                    idx_hbm.at[pl.ds(f * L, L), pl.ds(c * BCH, BCH)], idxc_v)

                @plsc.parallel_loop(0, NBG, 1, unroll=2)
                def group_body(g):
                    acc0 = jnp.zeros((NL,), jnp.float32)
                    acc1 = jnp.zeros((NL,), jnp.float32)
                    for l in range(0, L, 2):
                        i0 = idxc_v[l, pl.ds(g * NL, NL)]
                        i1 = idxc_v[l + 1, pl.ds(g * NL, NL)]
                        acc0 = acc0 + plsc.load_gather(tile_v, [i0])
                        acc1 = acc1 + plsc.load_gather(tile_v, [i1])
                    out_v[pl.ds(c * BCH + g * NL, NL)] = acc0 + acc1

                return carry2

            lax.fori_loop(0, NBC, chunk_body, 0)
            pltpu.sync_copy(out_v, out_hbm.at[p])
            return carry

        lax.fori_loop(0, ppw, pair_body, 0)

    return functools.partial(
        pl.kernel,
        out_type=jax.ShapeDtypeStruct((npairs, B), jnp.float32),
        mesh=plsc.VectorSubcoreMesh(core_axis_name="c", subcore_axis_name="s"),
        scratch_types=[
            pltpu.VMEM((V,), jnp.float32),
            pltpu.VMEM((L, BCH), jnp.int32),
            pltpu.VMEM((B,), jnp.float32),
        ],
        compiler_params=pltpu.CompilerParams(
            use_tc_tiling_on_sc=False, needs_layout_passes=False),
    )(body)


_calls = []
_off = 0
for _n in SPLITS:
    _calls.append((_off, _n, _make_call(_off, _n)))
    _off += _n


def kernel(sparse_indices, dense_feature, tables):
    tab_t = tables.transpose(0, 2, 1).reshape(F * D, V)
    idx_t = sparse_indices.transpose(0, 2, 1).reshape(F * L, B)
    parts = [c(tab_t[a:a + n], idx_t) for a, n, c in _calls]
    cols = [p.T for p in parts] + [dense_feature]
    return jnp.concatenate(cols, axis=1)


# speedup vs baseline: 10.8191x; 1.0208x over previous
import functools

import jax
import jax.numpy as jnp
from jax import lax
from jax.experimental import pallas as pl
from jax.experimental.pallas import tpu as pltpu
from jax.experimental.pallas import tpu_sc as plsc

F, B, L, V, D = 26, 4096, 20, 100000, 16
NC, NS, NL = 2, 16, 16
NW = NC * NS
NP = F * D
BCH = 512
NBC = B // BCH
NBG = BCH // NL
SPLITS = (96, 96, 96, 128)


def _make_call(start, npairs):
    ppw = npairs // NW

    def body(tab_hbm, idx_hbm, out_hbm, tile_v, idxc_v, out_v, sem_i):
        c_ax = lax.axis_index("c")
        s_ax = lax.axis_index("s")
        wid = s_ax * NC + c_ax

        def pair_body(k, carry):
            p = wid * ppw + k
            f = (start + p) // D
            pltpu.sync_copy(tab_hbm.at[p], tile_v)

            def i_slice(c):
                return idx_hbm.at[pl.ds(f * L, L), pl.ds(c * BCH, BCH)]

            pltpu.async_copy(i_slice(0), idxc_v.at[0], sem_i)

            def chunk_body(c, carry2):
                buf = lax.rem(c, 2)
                pltpu.make_async_copy(i_slice(c), idxc_v.at[buf], sem_i).wait()

                @pl.when(c + 1 < NBC)
                def _():
                    pltpu.async_copy(
                        i_slice(c + 1), idxc_v.at[1 - buf], sem_i)

                @plsc.parallel_loop(0, NBG, 1, unroll=2)
                def group_body(g):
                    acc0 = jnp.zeros((NL,), jnp.float32)
                    acc1 = jnp.zeros((NL,), jnp.float32)
                    for l in range(0, L, 2):
                        i0 = idxc_v[buf, l, pl.ds(g * NL, NL)]
                        i1 = idxc_v[buf, l + 1, pl.ds(g * NL, NL)]
                        acc0 = acc0 + plsc.load_gather(tile_v, [i0])
                        acc1 = acc1 + plsc.load_gather(tile_v, [i1])
                    out_v[pl.ds(c * BCH + g * NL, NL)] = acc0 + acc1

                return carry2

            lax.fori_loop(0, NBC, chunk_body, 0)
            pltpu.sync_copy(out_v, out_hbm.at[p])
            return carry

        lax.fori_loop(0, ppw, pair_body, 0)

    return functools.partial(
        pl.kernel,
        out_type=jax.ShapeDtypeStruct((npairs, B), jnp.float32),
        mesh=plsc.VectorSubcoreMesh(core_axis_name="c", subcore_axis_name="s"),
        scratch_types=[
            pltpu.VMEM((V,), jnp.float32),
            pltpu.VMEM((2, L, BCH), jnp.int32),
            pltpu.VMEM((B,), jnp.float32),
            pltpu.SemaphoreType.DMA,
        ],
        compiler_params=pltpu.CompilerParams(
            use_tc_tiling_on_sc=False, needs_layout_passes=False),
    )(body)


_calls = []
_off = 0
for _n in SPLITS:
    _calls.append((_off, _n, _make_call(_off, _n)))
    _off += _n


def kernel(sparse_indices, dense_feature, tables):
    tab_t = tables.transpose(0, 2, 1).reshape(F * D, V)
    idx_t = sparse_indices.transpose(0, 2, 1).reshape(F * L, B)
    parts = [c(tab_t[a:a + n], idx_t) for a, n, c in _calls]
    cols = [p.T for p in parts] + [dense_feature]
    return jnp.concatenate(cols, axis=1)


# speedup vs baseline: 10.8202x; 1.0001x over previous
import functools

import jax
import jax.numpy as jnp
from jax import lax
from jax.experimental import pallas as pl
from jax.experimental.pallas import tpu as pltpu
from jax.experimental.pallas import tpu_sc as plsc

F, B, L, V, D = 26, 4096, 20, 100000, 16
NC, NS, NL = 2, 16, 16
NW = NC * NS
NP = F * D
BCH = 512
NBC = B // BCH
NBG = BCH // NL
SPLITS = (96, 96, 96, 128)


def _make_call(start, npairs):
    ppw = npairs // NW

    def body(tab_hbm, idx_hbm, out_hbm, tile_v, idxc_v, out_v, sem_i, sem_t):
        c_ax = lax.axis_index("c")
        s_ax = lax.axis_index("s")
        wid = s_ax * NC + c_ax

        QV = V // 4

        def pair_body(k, carry):
            p = wid * ppw + k
            f = (start + p) // D
            for q in range(4):
                pltpu.async_copy(tab_hbm.at[p, pl.ds(q * QV, QV)],
                                 tile_v.at[pl.ds(q * QV, QV)], sem_t)
            for q in range(4):
                pltpu.make_async_copy(tab_hbm.at[p, pl.ds(q * QV, QV)],
                                      tile_v.at[pl.ds(q * QV, QV)],
                                      sem_t).wait()

            def i_slice(c):
                return idx_hbm.at[pl.ds(f * L, L), pl.ds(c * BCH, BCH)]

            pltpu.async_copy(i_slice(0), idxc_v.at[0], sem_i)

            def chunk_body(c, carry2):
                buf = lax.rem(c, 2)
                pltpu.make_async_copy(i_slice(c), idxc_v.at[buf], sem_i).wait()

                @pl.when(c + 1 < NBC)
                def _():
                    pltpu.async_copy(
                        i_slice(c + 1), idxc_v.at[1 - buf], sem_i)

                @plsc.parallel_loop(0, NBG, 1, unroll=2)
                def group_body(g):
                    acc0 = jnp.zeros((NL,), jnp.float32)
                    acc1 = jnp.zeros((NL,), jnp.float32)
                    for l in range(0, L, 2):
                        i0 = idxc_v[buf, l, pl.ds(g * NL, NL)]
                        i1 = idxc_v[buf, l + 1, pl.ds(g * NL, NL)]
                        acc0 = acc0 + plsc.load_gather(tile_v, [i0])
                        acc1 = acc1 + plsc.load_gather(tile_v, [i1])
                    out_v[pl.ds(c * BCH + g * NL, NL)] = acc0 + acc1

                return carry2

            lax.fori_loop(0, NBC, chunk_body, 0)
            pltpu.sync_copy(out_v, out_hbm.at[p])
            return carry

        lax.fori_loop(0, ppw, pair_body, 0)

    return functools.partial(
        pl.kernel,
        out_type=jax.ShapeDtypeStruct((npairs, B), jnp.float32),
        mesh=plsc.VectorSubcoreMesh(core_axis_name="c", subcore_axis_name="s"),
        scratch_types=[
            pltpu.VMEM((V,), jnp.float32),
            pltpu.VMEM((2, L, BCH), jnp.int32),
            pltpu.VMEM((B,), jnp.float32),
            pltpu.SemaphoreType.DMA,
            pltpu.SemaphoreType.DMA,
        ],
        compiler_params=pltpu.CompilerParams(
            use_tc_tiling_on_sc=False, needs_layout_passes=False),
    )(body)


_calls = []
_off = 0
for _n in SPLITS:
    _calls.append((_off, _n, _make_call(_off, _n)))
    _off += _n


def kernel(sparse_indices, dense_feature, tables):
    tab_t = tables.transpose(0, 2, 1).reshape(F * D, V)
    idx_t = sparse_indices.transpose(0, 2, 1).reshape(F * L, B)
    parts = [c(tab_t[a:a + n], idx_t) for a, n, c in _calls]
    cols = [p.T for p in parts] + [dense_feature]
    return jnp.concatenate(cols, axis=1)
